# baseline (speedup 1.0000x reference)
import functools

import jax
import jax.numpy as jnp
from jax import lax
from jax.experimental import pallas as pl
from jax.experimental.pallas import tpu as pltpu
from jax.experimental.pallas import tpu_sc as plsc

TAB_ROWS = 101
D = 64
ZR, ZC = 4096, 200
NC, NS = 2, 16
NW = NC * NS
ZR_PER_W = ZR // NW
CH_ZR = 2
N_CHUNKS = ZR_PER_W // CH_ZR
SPLITS = ((0, 104), (104, 96))


def _sc_gather(z, table):
    mesh = plsc.VectorSubcoreMesh(core_axis_name="c", subcore_axis_name="s")

    @functools.partial(
        pl.kernel,
        mesh=mesh,
        out_type=jax.ShapeDtypeStruct((ZR, ZC, D), jnp.float32),
        scratch_types=[
            pltpu.VMEM_SHARED((TAB_ROWS, D), jnp.float32),
            pltpu.VMEM((TAB_ROWS, D), jnp.float32),
            pltpu.VMEM((ZR_PER_W, ZC), jnp.int32),
            pltpu.VMEM((2, CH_ZR, ZC, D), jnp.float32),
            pltpu.SemaphoreType.DMA,
        ],
        compiler_params=pltpu.CompilerParams(use_tc_tiling_on_sc=False),
    )
    def k(z_hbm, table_hbm, out_hbm, table_sh, table_v, idx_v, rows_v, gsem):
        sid = lax.axis_index("s")
        wid = sid * NC + lax.axis_index("c")
        zr_base = wid * ZR_PER_W

        @pl.when(sid == 0)
        def _():
            pltpu.sync_copy(table_hbm, table_v)
            pltpu.sync_copy(table_v, table_sh)

        pltpu.sync_copy(z_hbm.at[pl.ds(zr_base, ZR_PER_W)], idx_v)
        plsc.subcore_barrier()

        def fire_gathers(c, buf):
            for r in range(CH_ZR):
                for off, ln in SPLITS:
                    pltpu.async_copy(
                        table_sh.at[idx_v.at[c * CH_ZR + r, pl.ds(off, ln)]],
                        rows_v.at[buf, r, pl.ds(off, ln)],
                        gsem,
                    )

        def wait_gathers(buf):
            for r in range(CH_ZR):
                for off, ln in SPLITS:
                    pltpu.make_async_copy(
                        table_sh.at[idx_v.at[r, pl.ds(off, ln)]],
                        rows_v.at[buf, r, pl.ds(off, ln)],
                        gsem,
                    ).wait()

        def copy_out(c, buf):
            pltpu.sync_copy(
                rows_v.at[buf],
                out_hbm.at[pl.ds(zr_base + c * CH_ZR, CH_ZR)],
            )

        fire_gathers(0, 0)

        def body(g, carry):
            for b in range(2):
                c = 2 * g + b
                fire_gathers(c + 1, 1 - b)
                wait_gathers(b)
                copy_out(c, b)
            return carry

        lax.fori_loop(0, (N_CHUNKS - 2) // 2, body, None)

        fire_gathers(N_CHUNKS - 1, 1)
        wait_gathers(0)
        copy_out(N_CHUNKS - 2, 0)
        wait_gathers(1)
        copy_out(N_CHUNKS - 1, 1)

    return k(z, table)


def kernel(z, table):
    return _sc_gather(z.astype(jnp.int32), table)
